# baseline (speedup 1.0000x reference)
import functools

import jax
import jax.numpy as jnp
from jax.experimental import pallas as pl
from jax.experimental.pallas import tpu as pltpu

_Q = 1024
_D = 16
_BK = 10000
_NKEYS = 100000


def _reduce_body(nsteps, yidx_ref, y_ref, xt2_ref, old_ref, m_ref, i_ref,
                 m_scr, i_scr, xsq_scr):
    step = pl.program_id(0)

    @pl.when(step == 0)
    def _():
        xt2 = xt2_ref[...]
        xsq_scr[0, :] = jnp.sum(xt2 * xt2, axis=0) * 0.25

    y = y_ref[...]
    y_sq = jnp.sum(y * y, axis=1, keepdims=True)
    prod = jnp.dot(y, xt2_ref[...],
                   preferred_element_type=jnp.float32)
    d2 = (y_sq + xsq_scr[0, :][None, :]) + prod
    bm = jnp.min(d2, axis=0)
    ba = jnp.argmin(d2, axis=0).astype(jnp.int32)
    base = step * _BK

    @pl.when(step == 0)
    def _():
        m_scr[0, :] = bm
        i_scr[0, :] = ba

    @pl.when(step > 0)
    def _():
        cur_m = m_scr[0, :]
        better = bm < cur_m
        m_scr[0, :] = jnp.where(better, bm, cur_m)
        i_scr[0, :] = jnp.where(better, ba + base, i_scr[0, :])

    @pl.when(step == nsteps - 1)
    def _():
        d = jnp.sqrt(jnp.maximum(m_scr[0, :], 0.0))
        m_ref[0, :] = jnp.minimum(d, old_ref[0, :])
        i_ref[0, :] = i_scr[0, :] + yidx_ref[0]


def _knn_reduce(y, xt2, old, y_idx_start, *, interpret=False):
    nkeys = y.shape[0]
    nsteps = nkeys // _BK
    yidx = jnp.asarray(y_idx_start, jnp.int32).reshape(1)
    return pl.pallas_call(
        functools.partial(_reduce_body, nsteps),
        grid=(nsteps,),
        in_specs=[
            pl.BlockSpec(memory_space=pltpu.SMEM),
            pl.BlockSpec((_BK, _D), lambda i: (i, 0)),
            pl.BlockSpec((_D, _Q), lambda i: (0, 0)),
            pl.BlockSpec((1, _Q), lambda i: (0, 0)),
        ],
        out_specs=[
            pl.BlockSpec((1, _Q), lambda i: (0, 0)),
            pl.BlockSpec((1, _Q), lambda i: (0, 0)),
        ],
        out_shape=[
            jax.ShapeDtypeStruct((1, _Q), jnp.float32),
            jax.ShapeDtypeStruct((1, _Q), jnp.int32),
        ],
        scratch_shapes=[
            pltpu.VMEM((1, _Q), jnp.float32),
            pltpu.VMEM((1, _Q), jnp.int32),
            pltpu.VMEM((1, _Q), jnp.float32),
        ],
        compiler_params=pltpu.CompilerParams(
            dimension_semantics=("arbitrary",),
        ),
        interpret=interpret,
    )(yidx, y, xt2, old)


def kernel(x, x_idx_start, y, y_idx_start, min_dists, nn_indices):
    xt2 = (-2.0 * x.reshape(_Q, _D)).T
    old = jax.lax.dynamic_slice(min_dists, (x_idx_start,), (_Q,))
    upd_d, upd_i = _knn_reduce(y, xt2, old.reshape(1, _Q), y_idx_start)
    min_dists_new = jax.lax.dynamic_update_slice(
        min_dists, upd_d.reshape(_Q), (x_idx_start,))
    nn_indices_new = jax.lax.dynamic_update_slice(
        nn_indices, upd_i.reshape(_Q).astype(nn_indices.dtype),
        (x_idx_start,))
    return (min_dists_new, nn_indices_new)


# speedup vs baseline: 1.0149x; 1.0149x over previous
import functools

import jax
import jax.numpy as jnp
from jax.experimental import pallas as pl
from jax.experimental.pallas import tpu as pltpu

_Q = 1024
_D = 16
_BK = 10000
_NKEYS = 100000


def _reduce_body(nsteps, y_ref, xt2_ref, m_ref, i_ref, m_scr, i_scr, xsq_scr):
    step = pl.program_id(0)

    @pl.when(step == 0)
    def _():
        xt2 = xt2_ref[...]
        xsq_scr[0, :] = jnp.sum(xt2 * xt2, axis=0) * 0.25

    y = y_ref[...]
    y_sq = jnp.sum(y * y, axis=1, keepdims=True)
    prod = jnp.dot(y, xt2_ref[...],
                   preferred_element_type=jnp.float32)
    d2 = (y_sq + xsq_scr[0, :][None, :]) + prod
    bm = jnp.min(d2, axis=0)
    ba = jnp.argmin(d2, axis=0).astype(jnp.int32)
    base = step * _BK

    @pl.when(step == 0)
    def _():
        m_scr[0, :] = bm
        i_scr[0, :] = ba

    @pl.when(step > 0)
    def _():
        cur_m = m_scr[0, :]
        better = bm < cur_m
        m_scr[0, :] = jnp.where(better, bm, cur_m)
        i_scr[0, :] = jnp.where(better, ba + base, i_scr[0, :])

    @pl.when(step == nsteps - 1)
    def _():
        m_ref[0, :] = m_scr[0, :]
        i_ref[0, :] = i_scr[0, :]


def _knn_reduce(y, xt2, *, interpret=False):
    nkeys = y.shape[0]
    nsteps = nkeys // _BK
    return pl.pallas_call(
        functools.partial(_reduce_body, nsteps),
        grid=(nsteps,),
        in_specs=[
            pl.BlockSpec((_BK, _D), lambda i: (i, 0)),
            pl.BlockSpec((_D, _Q), lambda i: (0, 0)),
        ],
        out_specs=[
            pl.BlockSpec((1, _Q), lambda i: (0, 0)),
            pl.BlockSpec((1, _Q), lambda i: (0, 0)),
        ],
        out_shape=[
            jax.ShapeDtypeStruct((1, _Q), jnp.float32),
            jax.ShapeDtypeStruct((1, _Q), jnp.int32),
        ],
        scratch_shapes=[
            pltpu.VMEM((1, _Q), jnp.float32),
            pltpu.VMEM((1, _Q), jnp.int32),
            pltpu.VMEM((1, _Q), jnp.float32),
        ],
        compiler_params=pltpu.CompilerParams(
            dimension_semantics=("arbitrary",),
        ),
        interpret=interpret,
    )(y, xt2)


def kernel(x, x_idx_start, y, y_idx_start, min_dists, nn_indices):
    xt2 = (-2.0 * x.reshape(_Q, _D)).T
    m, i = _knn_reduce(y, xt2)
    old = jax.lax.dynamic_slice(min_dists, (x_idx_start,), (_Q,))
    new_d = jnp.sqrt(jnp.maximum(m.reshape(_Q), 0.0))
    upd_d = jnp.minimum(new_d, old)
    upd_i = (i.reshape(_Q) + y_idx_start).astype(nn_indices.dtype)
    min_dists_new = jax.lax.dynamic_update_slice(min_dists, upd_d,
                                                 (x_idx_start,))
    nn_indices_new = jax.lax.dynamic_update_slice(nn_indices, upd_i,
                                                  (x_idx_start,))
    return (min_dists_new, nn_indices_new)
